# baseline (speedup 1.0000x reference)
import functools

import jax
import jax.numpy as jnp
from jax import lax
from jax.experimental import pallas as pl
from jax.experimental.pallas import tpu as pltpu
from jax.experimental.pallas import tpu_sc as plsc

_SEQ = 200
_BATCH = 4096
_EMBED = 64

_NC, _NS = 2, 16
_NW = _NC * _NS
_G = _BATCH // _NW
_NG = 4
_NCHUNK = _SEQ // _NG
_NBUF = 2
_NGROUP = _NCHUNK // _NBUF

_mesh = plsc.VectorSubcoreMesh(core_axis_name="c", subcore_axis_name="s")


@functools.partial(
    pl.kernel,
    mesh=_mesh,
    out_type=jax.ShapeDtypeStruct((_SEQ, _BATCH, _EMBED), jnp.float32),
    scratch_types=[
        pltpu.VMEM((_SEQ, _G), jnp.int32),
        pltpu.VMEM((_NBUF, _NG, _G, _EMBED), jnp.float32),
        pltpu.SemaphoreType.DMA,
        pltpu.SemaphoreType.DMA,
        pltpu.SemaphoreType.DMA,
        pltpu.SemaphoreType.DMA,
    ],
    compiler_params=pltpu.CompilerParams(use_tc_tiling_on_sc=False),
)
def _embed_gather(idx_hbm, table_hbm, out_hbm, idx_v, rows_v, g0, g1, w0, w1):
    gsem = [g0, g1]
    wsem = [w0, w1]
    wid = lax.axis_index("s") * _NC + lax.axis_index("c")
    col0 = wid * _G

    def gstart(ci, b):
        row0 = ci * _NG
        for j in range(_NG):
            pltpu.async_copy(
                table_hbm.at[idx_v.at[row0 + j]], rows_v.at[b, j], gsem[b])

    def gwait(ci, b):
        row0 = ci * _NG
        for j in range(_NG):
            pltpu.make_async_copy(
                table_hbm.at[idx_v.at[row0 + j]], rows_v.at[b, j],
                gsem[b]).wait()

    def wb(ci, b, sem):
        return pltpu.make_async_copy(
            rows_v.at[b],
            out_hbm.at[pl.ds(ci * _NG, _NG), pl.ds(col0, _G)],
            sem)

    pltpu.sync_copy(idx_hbm.at[:, pl.ds(col0, _G)], idx_v)

    for b in range(_NBUF):
        gstart(b, b)

    def group(g, carry):
        ci0 = g * _NBUF
        for b in range(_NBUF):
            gwait(ci0 + b, b)
            wb(ci0 + b, b, wsem[b]).start()
        for b in range(_NBUF):
            wb(ci0 + b, b, wsem[b]).wait()
            gstart(ci0 + _NBUF + b, b)
        return carry

    lax.fori_loop(0, _NGROUP - 1, group, 0)

    ci0 = (_NGROUP - 1) * _NBUF
    for b in range(_NBUF):
        gwait(ci0 + b, b)
        wb(ci0 + b, b, wsem[b]).start()
    for b in range(_NBUF):
        wb(ci0 + b, b, wsem[b]).wait()


def kernel(indices, table):
    return _embed_gather(indices.astype(jnp.int32), table)
